# Initial kernel scaffold; baseline (speedup 1.0000x reference)
import jax
import jax.numpy as jnp
from jax.experimental import pallas as pl


def kernel(x, W_down, b_down, W_up, b_up, expert_W1s, expert_b1s, expert_W2s, expert_b2s, W_shared1, b_shared1, W_shared2, b_shared2, W_router, b_router, W_core, b_core, expert_cost):
    raise NotImplementedError("write your pallas kernel here")


# speedup vs baseline: 4.0697x; 4.0697x over previous
import jax
import jax.numpy as jnp
from jax.experimental import pallas as pl

DIM = 1024
LATENT = 512
NE = 8
TOKENS = 2048
TB = 256
COST_LAMBDA = 5e-07
_SQRT_HALF = 0.7071067811865476


def _gelu(x):
    x = x.astype(jnp.float32)
    return x * 0.5 * (1.0 + jax.lax.erf(x * _SQRT_HALF))


def _bdot(a16, b16):
    return jax.lax.dot_general(
        a16, b16, (((1,), (1,)), ((), ())), preferred_element_type=jnp.float32)


def _stage1_body(x_ref, wdown_ref, wrouter_ref, brow_ref, wcore_ref, ws1_ref,
                 ws2_ref, bdown_ref, bcore_ref, bs1_ref, bs2_ref,
                 g_ref, wg_ref, counts_ref, sh_ref, core_ref):
    xb = x_ref[...]
    xb16 = xb.astype(jnp.bfloat16)

    h = _bdot(xb16, wdown_ref[...]) + bdown_ref[...]
    g = _gelu(h)
    g16 = g.astype(jnp.bfloat16)
    g_ref[...] = g16

    logits = jax.lax.dot_general(
        xb, wrouter_ref[...], (((1,), (1,)), ((), ())),
        preferred_element_type=jnp.float32,
        precision=jax.lax.Precision.HIGHEST) + brow_ref[...]
    mx = jnp.max(logits, axis=-1, keepdims=True)
    ex = jnp.exp(logits - mx)
    probs = ex / jnp.sum(ex, axis=-1, keepdims=True)

    iota = jax.lax.broadcasted_iota(jnp.int32, probs.shape, 1)
    m0 = jnp.max(probs, axis=-1, keepdims=True)
    e0 = jnp.min(jnp.where(probs >= m0, iota, NE), axis=-1, keepdims=True)
    oh0 = (iota == e0)
    probs1 = jnp.where(oh0, -1.0, probs)
    m1 = jnp.max(probs1, axis=-1, keepdims=True)
    e1 = jnp.min(jnp.where(probs1 >= m1, iota, NE), axis=-1, keepdims=True)
    oh1 = (iota == e1)

    ed = jnp.exp(m1 - m0)
    gate0 = 1.0 / (1.0 + ed)
    gate1 = ed * gate0

    w = gate0 * oh0.astype(jnp.float32) + gate1 * oh1.astype(jnp.float32)
    wg_ref[...] = jnp.concatenate(
        [w, jnp.broadcast_to(gate0, (TB, 4)), jnp.broadcast_to(gate1, (TB, 4))],
        axis=-1)

    a0 = jnp.max(oh0.astype(jnp.float32), axis=0, keepdims=True)
    a1 = jnp.max(oh1.astype(jnp.float32), axis=0, keepdims=True)
    cblk = jnp.broadcast_to(jnp.concatenate([a0, a1], axis=-1), (8, 2 * NE))

    @pl.when(pl.program_id(0) == 0)
    def _init():
        counts_ref[...] = cblk

    @pl.when(pl.program_id(0) > 0)
    def _acc():
        counts_ref[...] = jnp.maximum(counts_ref[...], cblk)

    core_ref[...] = _bdot(_gelu(xb).astype(jnp.bfloat16), wcore_ref[...]) \
        + bcore_ref[...]

    s1 = _gelu(_bdot(g16, ws1_ref[...]) + bs1_ref[...])
    sh_ref[...] = (_bdot(s1.astype(jnp.bfloat16), ws2_ref[...])
                   + bs2_ref[...]).astype(jnp.bfloat16)


def _stage2_body(counts_ref, b1p_ref, w2p_ref, b2c_ref, ca_ref):
    gb = _gelu(b1p_ref[...]).astype(jnp.bfloat16)
    rows = []
    for e in range(NE):
        ce = _bdot(gb[e:e + 1, :], w2p_ref[e]) + b2c_ref[e:e + 1, :]
        rows.append(ce)
    c = jnp.concatenate(rows, axis=0)
    act0 = counts_ref[0:1, 0:NE] > 0.5
    act1 = counts_ref[0:1, NE:2 * NE] > 0.5
    a0 = jnp.sum(jnp.where(act0.T, c, 0.0), axis=0, keepdims=True)
    a1 = jnp.sum(jnp.where(act1.T, c, 0.0), axis=0, keepdims=True)
    ca_ref[...] = jnp.concatenate(
        [c, a0, a1, jnp.zeros((6, LATENT), jnp.float32)], axis=0)


def _stage3_body(*refs):
    (g_ref, wg_ref, sh_ref, core_ref, ca_ref, wup_ref, bup_ref) = refs[:7]
    w1_refs = refs[7:7 + NE]
    b1_refs = refs[7 + NE:7 + 2 * NE]
    w2_refs = refs[7 + 2 * NE:7 + 3 * NE]
    b2_refs = refs[7 + 3 * NE:7 + 4 * NE]
    out_ref = refs[-1]

    g16 = g_ref[...]
    wg = wg_ref[...]
    moe = jnp.zeros((TB, LATENT), jnp.float32)
    for e in range(NE):
        z1 = _gelu(_bdot(g16, w1_refs[e][...]) + b1_refs[e][...])
        z2 = _bdot(z1.astype(jnp.bfloat16), w2_refs[e][...]) + b2_refs[e][...]
        ce = ca_ref[e:e + 1, :]
        moe = moe + wg[:, e:e + 1] * (z2 - ce)
    gate0 = wg[:, NE:NE + 1]
    gate1 = wg[:, NE + 4:NE + 5]
    acc = moe + gate0 * ca_ref[NE:NE + 1, :] + gate1 * ca_ref[NE + 1:NE + 2, :]
    t16 = (acc + 0.1 * sh_ref[...].astype(jnp.float32)).astype(jnp.bfloat16)
    out_ref[...] = _bdot(t16, wup_ref[...]) + bup_ref[...] + core_ref[...]


def kernel(x, W_down, b_down, W_up, b_up, expert_W1s, expert_b1s, expert_W2s,
           expert_b2s, W_shared1, b_shared1, W_shared2, b_shared2, W_router,
           b_router, W_core, b_core, expert_cost):
    f32 = jnp.float32
    bf16 = jnp.bfloat16
    X = x.reshape(TOKENS, DIM)
    nblk = TOKENS // TB

    brow = (b_router - COST_LAMBDA * expert_cost).reshape(1, NE)

    full = lambda shape: pl.BlockSpec(shape, lambda i: (0,) * len(shape))
    tb = lambda n: pl.BlockSpec((TB, n), lambda i: (i, 0))

    g16, wg, counts, sh16, core = pl.pallas_call(
        _stage1_body,
        grid=(nblk,),
        in_specs=[
            tb(DIM),
            full((LATENT, DIM)), full((NE, DIM)), full((1, NE)),
            full((DIM, DIM)), full((LATENT, LATENT)), full((LATENT, LATENT)),
            full((1, LATENT)), full((1, DIM)), full((1, LATENT)),
            full((1, LATENT)),
        ],
        out_specs=[
            tb(LATENT), tb(2 * NE),
            pl.BlockSpec((8, 2 * NE), lambda i: (0, 0)),
            tb(LATENT), tb(DIM),
        ],
        out_shape=[
            jax.ShapeDtypeStruct((TOKENS, LATENT), bf16),
            jax.ShapeDtypeStruct((TOKENS, 2 * NE), f32),
            jax.ShapeDtypeStruct((8, 2 * NE), f32),
            jax.ShapeDtypeStruct((TOKENS, LATENT), bf16),
            jax.ShapeDtypeStruct((TOKENS, DIM), f32),
        ],
    )(X, W_down.astype(bf16), W_router, brow, W_core.astype(bf16),
      W_shared1.astype(bf16), W_shared2.astype(bf16),
      b_down.reshape(1, LATENT), b_core.reshape(1, DIM),
      b_shared1.reshape(1, LATENT), b_shared2.reshape(1, LATENT))

    hmax = max(w.shape[0] for w in expert_W1s)
    b1p = jnp.zeros((NE, hmax), f32)
    w2p = jnp.zeros((NE, LATENT, hmax), bf16)
    for e in range(NE):
        hd = expert_b1s[e].shape[0]
        b1p = b1p.at[e, :hd].set(expert_b1s[e])
        w2p = w2p.at[e, :, :hd].set(expert_W2s[e].astype(bf16))
    b2c = jnp.stack(expert_b2s, axis=0)

    ca = pl.pallas_call(
        _stage2_body,
        grid=(1,),
        in_specs=[full((8, 2 * NE)), full((NE, hmax)),
                  full((NE, LATENT, hmax)), full((NE, LATENT))],
        out_specs=full((2 * NE, LATENT)),
        out_shape=jax.ShapeDtypeStruct((2 * NE, LATENT), f32),
    )(counts, b1p, w2p, b2c)

    in_specs3 = [tb(LATENT), tb(2 * NE), tb(LATENT), tb(DIM),
                 full((2 * NE, LATENT)), full((DIM, LATENT)), full((1, DIM))]
    args3 = [g16, wg, sh16, core, ca, W_up.astype(bf16), b_up.reshape(1, DIM)]
    for e in range(NE):
        hd = expert_W1s[e].shape[0]
        in_specs3.append(full((hd, LATENT)))
        args3.append(expert_W1s[e].astype(bf16))
    for e in range(NE):
        hd = expert_b1s[e].shape[0]
        in_specs3.append(full((1, hd)))
        args3.append(expert_b1s[e].reshape(1, hd))
    for e in range(NE):
        hd = expert_W2s[e].shape[1]
        in_specs3.append(full((LATENT, hd)))
        args3.append(expert_W2s[e].astype(bf16))
    for e in range(NE):
        in_specs3.append(full((1, LATENT)))
        args3.append(expert_b2s[e].reshape(1, LATENT))

    out = pl.pallas_call(
        _stage3_body,
        grid=(nblk,),
        in_specs=in_specs3,
        out_specs=tb(DIM),
        out_shape=jax.ShapeDtypeStruct((TOKENS, DIM), f32),
    )(*args3)

    return out.reshape(x.shape)


# speedup vs baseline: 5.4137x; 1.3303x over previous
import jax
import jax.numpy as jnp
from jax.experimental import pallas as pl
from jax.experimental.pallas import tpu as pltpu

DIM = 1024
LATENT = 512
NE = 8
TOKENS = 2048
TB = 512
COST_LAMBDA = 5e-07
_SQRT_HALF = 0.7071067811865476


def _gelu(x):
    x = x.astype(jnp.float32)
    return x * 0.5 * (1.0 + jax.lax.erf(x * _SQRT_HALF))


def _bdot(a16, b16):
    return jax.lax.dot_general(
        a16, b16, (((1,), (1,)), ((), ())), preferred_element_type=jnp.float32)


def _stage1_body(x_ref, wdown_ref, wrhi_ref, wrlo_ref, brow_ref, wcore_ref,
                 ws1_ref, ws2_ref, bdown_ref, bcore_ref, bs1_ref, bs2_ref,
                 g_ref, wg_ref, counts_ref, sh_ref, core_ref):
    xb = x_ref[...]
    xb16 = xb.astype(jnp.bfloat16)
    xlo16 = (xb - xb16.astype(jnp.float32)).astype(jnp.bfloat16)

    h = _bdot(xb16, wdown_ref[...]) + bdown_ref[...]
    g = _gelu(h)
    g16 = g.astype(jnp.bfloat16)
    g_ref[...] = g16

    logits = (_bdot(xb16, wrhi_ref[...])
              + (_bdot(xb16, wrlo_ref[...]) + _bdot(xlo16, wrhi_ref[...]))
              + brow_ref[...])
    mx = jnp.max(logits, axis=-1, keepdims=True)
    ex = jnp.exp(logits - mx)
    probs = ex / jnp.sum(ex, axis=-1, keepdims=True)

    iota = jax.lax.broadcasted_iota(jnp.int32, probs.shape, 1)
    m0 = jnp.max(probs, axis=-1, keepdims=True)
    e0 = jnp.min(jnp.where(probs >= m0, iota, NE), axis=-1, keepdims=True)
    oh0 = (iota == e0)
    probs1 = jnp.where(oh0, -1.0, probs)
    m1 = jnp.max(probs1, axis=-1, keepdims=True)
    e1 = jnp.min(jnp.where(probs1 >= m1, iota, NE), axis=-1, keepdims=True)
    oh1 = (iota == e1)

    ed = jnp.exp(m1 - m0)
    gate0 = 1.0 / (1.0 + ed)
    gate1 = ed * gate0

    w = gate0 * oh0.astype(jnp.float32) + gate1 * oh1.astype(jnp.float32)
    wg_ref[...] = jnp.concatenate(
        [w, jnp.broadcast_to(gate0, (TB, 4)), jnp.broadcast_to(gate1, (TB, 4))],
        axis=-1)

    a0 = jnp.max(oh0.astype(jnp.float32), axis=0, keepdims=True)
    a1 = jnp.max(oh1.astype(jnp.float32), axis=0, keepdims=True)
    cblk = jnp.broadcast_to(jnp.concatenate([a0, a1], axis=-1), (8, 2 * NE))

    @pl.when(pl.program_id(0) == 0)
    def _init():
        counts_ref[...] = cblk

    @pl.when(pl.program_id(0) > 0)
    def _acc():
        counts_ref[...] = jnp.maximum(counts_ref[...], cblk)

    core_ref[...] = _bdot(_gelu(xb).astype(jnp.bfloat16), wcore_ref[...]) \
        + bcore_ref[...]

    s1 = _gelu(_bdot(g16, ws1_ref[...]) + bs1_ref[...])
    sh_ref[...] = (_bdot(s1.astype(jnp.bfloat16), ws2_ref[...])
                   + bs2_ref[...]).astype(jnp.bfloat16)


def _stage3_body(*refs):
    (g_ref, wg_ref, sh_ref, core_ref, counts_ref, wup_ref, bup_ref) = refs[:7]
    w1_refs = refs[7:7 + NE]
    b1_refs = refs[7 + NE:7 + 2 * NE]
    w2_refs = refs[7 + 2 * NE:7 + 3 * NE]
    b2_refs = refs[7 + 3 * NE:7 + 4 * NE]
    out_ref = refs[-2]
    ca_ref = refs[-1]

    @pl.when(pl.program_id(0) == 0)
    def _constants():
        for e in range(NE):
            gb = _gelu(b1_refs[e][...]).astype(jnp.bfloat16)
            ca_ref[e:e + 1, :] = _bdot(gb, w2_refs[e][...]) + b2_refs[e][...]
        c = ca_ref[0:NE, :]
        act0 = counts_ref[0:1, 0:NE] > 0.5
        act1 = counts_ref[0:1, NE:2 * NE] > 0.5
        ca_ref[NE:NE + 1, :] = jnp.sum(jnp.where(act0.T, c, 0.0), axis=0,
                                       keepdims=True)
        ca_ref[NE + 1:NE + 2, :] = jnp.sum(jnp.where(act1.T, c, 0.0), axis=0,
                                           keepdims=True)

    g16 = g_ref[...]
    wg = wg_ref[...]
    moe = jnp.zeros((TB, LATENT), jnp.float32)
    for e in range(NE):
        z1 = _gelu(_bdot(g16, w1_refs[e][...]) + b1_refs[e][...])
        z2 = _bdot(z1.astype(jnp.bfloat16), w2_refs[e][...]) + b2_refs[e][...]
        moe = moe + wg[:, e:e + 1] * (z2 - ca_ref[e:e + 1, :])
    gate0 = wg[:, NE:NE + 1]
    gate1 = wg[:, NE + 4:NE + 5]
    acc = (moe + gate0 * ca_ref[NE:NE + 1, :]
           + gate1 * ca_ref[NE + 1:NE + 2, :])
    t16 = (acc + 0.1 * sh_ref[...].astype(jnp.float32)).astype(jnp.bfloat16)
    out_ref[...] = _bdot(t16, wup_ref[...]) + bup_ref[...] + core_ref[...]


def kernel(x, W_down, b_down, W_up, b_up, expert_W1s, expert_b1s, expert_W2s,
           expert_b2s, W_shared1, b_shared1, W_shared2, b_shared2, W_router,
           b_router, W_core, b_core, expert_cost):
    f32 = jnp.float32
    bf16 = jnp.bfloat16
    X = x.reshape(TOKENS, DIM)
    nblk = TOKENS // TB

    brow = (b_router - COST_LAMBDA * expert_cost).reshape(1, NE)
    wr_hi = W_router.astype(bf16)
    wr_lo = (W_router - wr_hi.astype(f32)).astype(bf16)

    full = lambda shape: pl.BlockSpec(shape, lambda i: (0,) * len(shape))
    tb = lambda n: pl.BlockSpec((TB, n), lambda i: (i, 0))

    g16, wg, counts, sh16, core = pl.pallas_call(
        _stage1_body,
        grid=(nblk,),
        in_specs=[
            tb(DIM),
            full((LATENT, DIM)), full((NE, DIM)), full((NE, DIM)),
            full((1, NE)),
            full((DIM, DIM)), full((LATENT, LATENT)), full((LATENT, LATENT)),
            full((1, LATENT)), full((1, DIM)), full((1, LATENT)),
            full((1, LATENT)),
        ],
        out_specs=[
            tb(LATENT), tb(2 * NE),
            pl.BlockSpec((8, 2 * NE), lambda i: (0, 0)),
            tb(LATENT), tb(DIM),
        ],
        out_shape=[
            jax.ShapeDtypeStruct((TOKENS, LATENT), bf16),
            jax.ShapeDtypeStruct((TOKENS, 2 * NE), f32),
            jax.ShapeDtypeStruct((8, 2 * NE), f32),
            jax.ShapeDtypeStruct((TOKENS, LATENT), bf16),
            jax.ShapeDtypeStruct((TOKENS, DIM), f32),
        ],
    )(X, W_down.astype(bf16), wr_hi, wr_lo, brow, W_core.astype(bf16),
      W_shared1.astype(bf16), W_shared2.astype(bf16),
      b_down.reshape(1, LATENT), b_core.reshape(1, DIM),
      b_shared1.reshape(1, LATENT), b_shared2.reshape(1, LATENT))

    in_specs3 = [tb(LATENT), tb(2 * NE), tb(LATENT), tb(DIM),
                 full((8, 2 * NE)), full((DIM, LATENT)), full((1, DIM))]
    args3 = [g16, wg, sh16, core, counts, W_up.astype(bf16),
             b_up.reshape(1, DIM)]
    for e in range(NE):
        hd = expert_W1s[e].shape[0]
        in_specs3.append(full((hd, LATENT)))
        args3.append(expert_W1s[e].astype(bf16))
    for e in range(NE):
        hd = expert_b1s[e].shape[0]
        in_specs3.append(full((1, hd)))
        args3.append(expert_b1s[e].reshape(1, hd))
    for e in range(NE):
        hd = expert_W2s[e].shape[1]
        in_specs3.append(full((LATENT, hd)))
        args3.append(expert_W2s[e].astype(bf16))
    for e in range(NE):
        in_specs3.append(full((1, LATENT)))
        args3.append(expert_b2s[e].reshape(1, LATENT))

    out = pl.pallas_call(
        _stage3_body,
        grid=(nblk,),
        in_specs=in_specs3,
        out_specs=tb(DIM),
        out_shape=jax.ShapeDtypeStruct((TOKENS, DIM), f32),
        scratch_shapes=[pltpu.VMEM((2 * NE, LATENT), f32)],
    )(*args3)

    return out.reshape(x.shape)
